# baseline (speedup 1.0000x reference)
import functools

import jax
import jax.numpy as jnp
import numpy as np
from jax.experimental import pallas as pl
from jax.experimental.pallas import tpu as pltpu

N = 10000
E = 320000
D_EDGE = 26
H = 128
TBL = 256


def _leaky(x):
    return jnp.where(x >= 0, x, 0.01 * x)


def _mm(x, w):
    return jax.lax.dot_general(x, w, (((1,), (0,)), ((), ())),
                               preferred_element_type=jnp.float32)


def _softplus(x):
    return jnp.maximum(x, 0.0) + jnp.log1p(jnp.exp(-jnp.abs(x)))


def _embed_body(xf, w1, b1, w2, b2, w3, b3, h_out):
    h = _leaky(_mm(xf[...], w1[...]) + b1[...])
    h = _leaky(_mm(h, w2[...]) + b2[...])
    h_out[...] = _leaky(_mm(h, w3[...]) + b3[...])


def _embed(xf, w1, b1, w2, b2, w3, b3, blk=2000):
    grid = N // blk
    full = lambda a: pl.BlockSpec(a.shape, lambda i: (0, 0))
    return pl.pallas_call(
        _embed_body,
        grid=(grid,),
        in_specs=[pl.BlockSpec((blk, 128), lambda i: (i, 0)),
                  full(w1), full(b1), full(w2), full(b2), full(w3), full(b3)],
        out_specs=pl.BlockSpec((blk, 128), lambda i: (i, 0)),
        out_shape=jax.ShapeDtypeStruct((N, 128), jnp.float32),
    )(xf, w1, b1, w2, b2, w3, b3)


def _epre_body(el, we, be, out):
    j = jax.lax.broadcasted_iota(jnp.int32, (1, 32), 1).astype(jnp.float32) * 0.2
    ea = jnp.exp(-((el[...] - j) ** 2) * 25.0)
    out[...] = _mm(ea, we[...]) + be[...]


def _epre(el, we, be, blk=4000):
    grid = E // blk
    return pl.pallas_call(
        _epre_body,
        grid=(grid,),
        in_specs=[pl.BlockSpec((blk, 1), lambda i: (i, 0)),
                  pl.BlockSpec(we.shape, lambda i: (0, 0)),
                  pl.BlockSpec(be.shape, lambda i: (0, 0))],
        out_specs=pl.BlockSpec((blk, 768), lambda i: (i, 0)),
        out_shape=jax.ShapeDtypeStruct((E, 768), jnp.float32),
    )(el, we, be)


def _tables_body(h, wd, ws, h_out, td_out, ts_out):
    hv = h[...]
    h_out[...] = hv
    td_out[...] = _mm(hv, wd[...])
    ts_out[...] = _mm(hv, ws[...])


def _update_tables_body(h, acc, wd, ws, h_out, td_out, ts_out, *, leaky):
    hv = h[...] + acc[...]
    if leaky:
        hv = _leaky(hv)
    h_out[...] = hv
    td_out[...] = _mm(hv, wd[...])
    ts_out[...] = _mm(hv, ws[...])


def _tables(h, wd, ws, acc=None, leaky=True, blk=2000):
    grid = N // blk
    full = lambda a: pl.BlockSpec(a.shape, lambda i: (0, 0))
    row = lambda d: pl.BlockSpec((blk, d), lambda i: (i, 0))
    out_shape = (jax.ShapeDtypeStruct((N, 128), jnp.float32),
                 jax.ShapeDtypeStruct((N, TBL), jnp.float32),
                 jax.ShapeDtypeStruct((N, TBL), jnp.float32))
    out_specs = (row(128), row(TBL), row(TBL))
    if acc is None:
        return pl.pallas_call(
            _tables_body, grid=(grid,),
            in_specs=[row(128), full(wd), full(ws)],
            out_specs=out_specs, out_shape=out_shape,
        )(h, wd, ws)
    return pl.pallas_call(
        functools.partial(_update_tables_body, leaky=leaky), grid=(grid,),
        in_specs=[row(128), row(128), full(wd), full(ws)],
        out_specs=out_specs, out_shape=out_shape,
    )(h, acc, wd, ws)


def _elem_body(g, m_out):
    gv = g[...]
    f = gv[:, :H]
    s = gv[:, H:]
    m_out[...] = jax.nn.sigmoid(f) * _softplus(s)


def _elem(g, blk=8000):
    grid = E // blk
    return pl.pallas_call(
        _elem_body,
        grid=(grid,),
        in_specs=[pl.BlockSpec((blk, TBL), lambda i: (i, 0))],
        out_specs=pl.BlockSpec((blk, H), lambda i: (i, 0)),
        out_shape=jax.ShapeDtypeStruct((E, H), jnp.float32),
    )(g)


def _head_body(h, acc, wd1, bd1, wd2, bd2, out):
    hv = h[...] + acc[...]
    t = jnp.maximum(_mm(hv, wd1[...]) + bd1[...], 0.0)
    out[...] = _mm(t, wd2[...]) + bd2[...]


def _head(h, acc, wd1, bd1, wd2, bd2, blk=2000):
    grid = N // blk
    full = lambda a: pl.BlockSpec(a.shape, lambda i: (0, 0))
    return pl.pallas_call(
        _head_body,
        grid=(grid,),
        in_specs=[pl.BlockSpec((blk, 128), lambda i: (i, 0)),
                  pl.BlockSpec((blk, 128), lambda i: (i, 0)),
                  full(wd1), full(bd1), full(wd2), full(bd2)],
        out_specs=pl.BlockSpec((blk, 3), lambda i: (i, 0)),
        out_shape=jax.ShapeDtypeStruct((N, 3), jnp.float32),
    )(h, acc, wd1, bd1, wd2, bd2)


def kernel(x, edge_index, edge_length, forces_stack, forces_norm,
           W_e1, b_e1, W_e2, b_e2, W_e3, b_e3,
           Wf1, bf1, Ws1, bs1, Wf2, bf2, Ws2, bs2, Wf3, bf3, Ws3, bs3,
           W_d1, b_d1, W_d2, b_d2):
    xf = jnp.concatenate([x, forces_stack, forces_norm], axis=1)
    h0 = _embed(xf, W_e1, b_e1.reshape(1, -1), W_e2, b_e2.reshape(1, -1),
                W_e3, b_e3.reshape(1, -1))

    def _we(Wf, Ws):
        w = jnp.concatenate([Wf[256:], Ws[256:]], axis=1)
        return jnp.pad(w, ((0, 32 - D_EDGE), (0, 0)))
    we_all = jnp.concatenate([_we(Wf1, Ws1), _we(Wf2, Ws2), _we(Wf3, Ws3)],
                             axis=1)
    be_all = jnp.concatenate([bf1, bs1, bf2, bs2, bf3, bs3]).reshape(1, 768)
    epre = _epre(edge_length.reshape(E, 1), we_all, be_all)

    dst = edge_index[1]
    src = edge_index[0]

    def _wd(Wf, Ws):
        return (jnp.concatenate([Wf[:128], Ws[:128]], axis=1),
                jnp.concatenate([Wf[128:256], Ws[128:256]], axis=1))

    h = h0
    acc = None
    layer_w = [(Wf1, Ws1), (Wf2, Ws2), (Wf3, Ws3)]
    for l in range(3):
        wd, ws = _wd(*layer_w[l])
        h, td, ts = _tables(h, wd, ws, acc=acc, leaky=True)
        g = jnp.take(td, dst, axis=0) + jnp.take(ts, src, axis=0) \
            + jax.lax.dynamic_slice_in_dim(epre, l * TBL, TBL, axis=1)
        m = _elem(g)
        acc = jax.ops.segment_sum(m, dst, num_segments=N)

    return _head(h, acc, W_d1, b_d1.reshape(1, -1), W_d2, b_d2.reshape(1, -1))


# speedup vs baseline: 3.6808x; 3.6808x over previous
import functools

import jax
import jax.numpy as jnp
from jax import lax
from jax.experimental import pallas as pl
from jax.experimental.pallas import tpu as pltpu
from jax.experimental.pallas import tpu_sc as plsc

N = 10000
E = 320000
D_EDGE = 26
H = 128

_NC, _NS = 2, 16
_NW = _NC * _NS
_EPW = E // _NW
_CG = 40
_NCHG = _EPW // _CG
_EPT = E // _NS
_CS = 80
_NCHS = _EPT // _CS
_NB = 5
_HN = 5120
_TRASH = _HN
_ACCR = _HN + 8
_RPT = _HN // _NS


def _leaky(x):
    return jnp.where(x >= 0, x, 0.01 * x)


def _mm(x, w):
    return jax.lax.dot_general(x, w, (((1,), (0,)), ((), ())),
                               preferred_element_type=jnp.float32)


def _softplus(x):
    return jnp.maximum(x, 0.0) + jnp.log1p(jnp.exp(-jnp.abs(x)))


def _pack_bf16(f, s):
    uf = lax.bitcast_convert_type(f, jnp.int32)
    uf = uf + 0x7FFF + (lax.shift_right_logical(uf, 16) & 1)
    us = lax.bitcast_convert_type(s, jnp.int32)
    us = us + 0x7FFF + (lax.shift_right_logical(us, 16) & 1)
    return (us & jnp.int32(-65536)) | lax.shift_right_logical(uf, 16)


def _unpack_f(w):
    return lax.bitcast_convert_type(lax.shift_left(w, 16), jnp.float32)


def _unpack_s(w):
    return lax.bitcast_convert_type(w & jnp.int32(-65536), jnp.float32)


def _embed_body(xf, w1, b1, w2, b2, w3, b3, h_out):
    h = _leaky(_mm(xf[...], w1[...]) + b1[...])
    h = _leaky(_mm(h, w2[...]) + b2[...])
    h_out[...] = _leaky(_mm(h, w3[...]) + b3[...])


def _embed(xf, w1, b1, w2, b2, w3, b3, blk=2000):
    grid = N // blk
    full = lambda a: pl.BlockSpec(a.shape, lambda i: (0, 0))
    return pl.pallas_call(
        _embed_body,
        grid=(grid,),
        in_specs=[pl.BlockSpec((blk, 128), lambda i: (i, 0)),
                  full(w1), full(b1), full(w2), full(b2), full(w3), full(b3)],
        out_specs=pl.BlockSpec((blk, 128), lambda i: (i, 0)),
        out_shape=jax.ShapeDtypeStruct((N, 128), jnp.float32),
    )(xf, w1, b1, w2, b2, w3, b3)


def _epre_body(el, we, be, out):
    j = jax.lax.broadcasted_iota(jnp.int32, (1, 32), 1).astype(jnp.float32) * 0.2
    ea = jnp.exp(-((el[...] - j) ** 2) * 25.0)
    out[...] = (_mm(ea, we[...]) + be[...]).astype(jnp.bfloat16)


def _epre(el, we, be, blk=4000):
    grid = E // blk
    return pl.pallas_call(
        _epre_body,
        grid=(grid,),
        in_specs=[pl.BlockSpec((blk, 1), lambda i: (i, 0)),
                  pl.BlockSpec(we.shape, lambda i: (0, 0)),
                  pl.BlockSpec(be.shape, lambda i: (0, 0))],
        out_specs=pl.BlockSpec((blk, 768), lambda i: (i, 0)),
        out_shape=jax.ShapeDtypeStruct((E, 768), jnp.bfloat16),
    )(el, we, be)


def _tables_body(h, wdf, wds, wsf, wss, h_out, td_out, ts_out):
    hv = h[...]
    h_out[...] = hv
    td_out[...] = _pack_bf16(_mm(hv, wdf[...]), _mm(hv, wds[...]))
    ts_out[...] = _pack_bf16(_mm(hv, wsf[...]), _mm(hv, wss[...]))


def _update_tables_body(h, a, wdf, wds, wsf, wss, h_out, td_out, ts_out,
                        *, leaky):
    hv = h[...] + a[...]
    if leaky:
        hv = _leaky(hv)
    h_out[...] = hv
    td_out[...] = _pack_bf16(_mm(hv, wdf[...]), _mm(hv, wds[...]))
    ts_out[...] = _pack_bf16(_mm(hv, wsf[...]), _mm(hv, wss[...]))


def _tables(h, ws4, acc=None, leaky=True, blk=2000):
    grid = N // blk
    full = lambda a: pl.BlockSpec(a.shape, lambda i: (0, 0))
    row = pl.BlockSpec((blk, 128), lambda i: (i, 0))
    out_shape = (jax.ShapeDtypeStruct((N, 128), jnp.float32),
                 jax.ShapeDtypeStruct((N, 128), jnp.int32),
                 jax.ShapeDtypeStruct((N, 128), jnp.int32))
    out_specs = (row, row, row)
    wspecs = [full(w) for w in ws4]
    if acc is None:
        return pl.pallas_call(
            _tables_body, grid=(grid,),
            in_specs=[row] + wspecs,
            out_specs=out_specs, out_shape=out_shape,
        )(h, *ws4)
    return pl.pallas_call(
        functools.partial(_update_tables_body, leaky=leaky), grid=(grid,),
        in_specs=[row, row] + wspecs,
        out_specs=out_specs, out_shape=out_shape,
    )(h, acc, *ws4)


def _elem_body(gd, gs, epf, eps, m_out):
    wd = gd[...]
    ws = gs[...]
    f = (_unpack_f(wd) + _unpack_f(ws) + epf[...].astype(jnp.float32))
    s = (_unpack_s(wd) + _unpack_s(ws) + eps[...].astype(jnp.float32))
    m_out[...] = jax.nn.sigmoid(f) * _softplus(s)


def _elem(gd, gs, epre, layer, blk=8000):
    grid = E // blk
    row = pl.BlockSpec((blk, 128), lambda i: (i, 0))
    col = lambda j: pl.BlockSpec((blk, H), lambda i: (i, j))
    return pl.pallas_call(
        _elem_body,
        grid=(grid,),
        in_specs=[row, row, col(2 * layer), col(2 * layer + 1)],
        out_specs=row,
        out_shape=jax.ShapeDtypeStruct((E, H), jnp.float32),
    )(gd, gs, epre, epre)


def _head_body(h, a, wd1, bd1, wd2, bd2, out):
    hv = h[...] + a[...]
    t = jnp.maximum(_mm(hv, wd1[...]) + bd1[...], 0.0)
    out[...] = _mm(t, wd2[...]) + bd2[...]


def _head(h, acc, wd1, bd1, wd2, bd2, blk=2000):
    grid = N // blk
    full = lambda a: pl.BlockSpec(a.shape, lambda i: (0, 0))
    row = pl.BlockSpec((blk, 128), lambda i: (i, 0))
    return pl.pallas_call(
        _head_body,
        grid=(grid,),
        in_specs=[row, row, full(wd1), full(bd1), full(wd2), full(bd2)],
        out_specs=pl.BlockSpec((blk, 3), lambda i: (i, 0)),
        out_shape=jax.ShapeDtypeStruct((N, 3), jnp.float32),
    )(h, acc, wd1, bd1, wd2, bd2)


def _sc_gather(td, ts, dst, src):
    mesh = plsc.VectorSubcoreMesh(core_axis_name="c", subcore_axis_name="s")
    scratch = [
        pltpu.VMEM((_EPW,), jnp.int32),
        pltpu.VMEM((_EPW,), jnp.int32),
    ]
    scratch += [pltpu.VMEM((_CG, 128), jnp.int32) for _ in range(2 * _NB)]
    scratch += [pltpu.SemaphoreType.DMA((_NB,)) for _ in range(4)]

    @functools.partial(
        pl.kernel,
        out_type=(jax.ShapeDtypeStruct((E, 128), jnp.int32),
                  jax.ShapeDtypeStruct((E, 128), jnp.int32)),
        mesh=mesh,
        scratch_types=scratch,
    )
    def k(td_hbm, ts_hbm, dst_hbm, src_hbm, gd_hbm, gs_hbm,
          idx_d, idx_s, *bufsem):
        bufs_d = bufsem[0:_NB]
        bufs_s = bufsem[_NB:2 * _NB]
        gsem_d, gsem_s, wsem_d, wsem_s = bufsem[2 * _NB:2 * _NB + 4]
        c = lax.axis_index("c")
        s = lax.axis_index("s")
        wid = c * _NS + s
        eb = pl.multiple_of(wid * _EPW, _EPW)
        pltpu.sync_copy(dst_hbm.at[pl.ds(eb, _EPW)], idx_d)
        pltpu.sync_copy(src_hbm.at[pl.ds(eb, _EPW)], idx_s)

        def g_start(ch, b):
            off = pl.multiple_of(ch * _CG, 8)
            pltpu.make_async_copy(
                td_hbm.at[idx_d.at[pl.ds(off, _CG)]], bufs_d[b],
                gsem_d.at[b]).start()
            pltpu.make_async_copy(
                ts_hbm.at[idx_s.at[pl.ds(off, _CG)]], bufs_s[b],
                gsem_s.at[b]).start()

        def g_wait(b):
            pltpu.make_async_copy(td_hbm.at[pl.ds(0, _CG)], bufs_d[b],
                                  gsem_d.at[b]).wait()
            pltpu.make_async_copy(ts_hbm.at[pl.ds(0, _CG)], bufs_s[b],
                                  gsem_s.at[b]).wait()

        def w_start(ch, b):
            off = pl.multiple_of(eb + ch * _CG, 8)
            pltpu.make_async_copy(
                bufs_d[b], gd_hbm.at[pl.ds(off, _CG)], wsem_d.at[b]).start()
            pltpu.make_async_copy(
                bufs_s[b], gs_hbm.at[pl.ds(off, _CG)], wsem_s.at[b]).start()

        def w_wait(b):
            pltpu.make_async_copy(bufs_d[b], gd_hbm.at[pl.ds(0, _CG)],
                                  wsem_d.at[b]).wait()
            pltpu.make_async_copy(bufs_s[b], gs_hbm.at[pl.ds(0, _CG)],
                                  wsem_s.at[b]).wait()

        g_start(0, 0)
        g_start(1, 1)

        def outer(i, _):
            kk = i * _NB
            for b in range(_NB):
                ch = kk + b
                b2 = (b + 2) % _NB

                @pl.when(ch + 2 < _NCHG)
                def _():
                    @pl.when(ch >= 3)
                    def _():
                        w_wait(b2)
                    g_start(ch + 2, b2)

                g_wait(b)
                w_start(ch, b)
            return _

        lax.fori_loop(0, _NCHG // _NB, outer, None)
        for b in ((_NCHG - 3) % _NB, (_NCHG - 2) % _NB, (_NCHG - 1) % _NB):
            w_wait(b)

    return k(td, ts, dst, src)


def _sc_scatter(m, dst3, zeros):
    mesh = plsc.VectorSubcoreMesh(core_axis_name="c", subcore_axis_name="s")
    scratch = [pltpu.VMEM((_NCHS, _CS), jnp.int32)]
    scratch += [pltpu.VMEM((_CS, 128), jnp.float32) for _ in range(_NB)]
    scratch += [pltpu.SemaphoreType.DMA((_NB,)),
                pltpu.VMEM_SHARED((_ACCR, 128), jnp.float32)]

    @functools.partial(
        pl.kernel,
        out_type=jax.ShapeDtypeStruct((2, _HN, 128), jnp.float32),
        mesh=mesh,
        scratch_types=scratch,
    )
    def k(m_hbm, dst_hbm, z_hbm, out_hbm, idx3, *rest):
        mbufs = rest[0:_NB]
        msem = rest[_NB]
        acc = rest[_NB + 1]
        c = lax.axis_index("c")
        s = lax.axis_index("s")
        eb = pl.multiple_of(s * _EPT, 8)
        rb = pl.multiple_of(s * _RPT, 8)
        pltpu.sync_copy(z_hbm.at[pl.ds(rb, _RPT)], acc.at[pl.ds(rb, _RPT)])
        pltpu.sync_copy(dst_hbm.at[s], idx3)

        base = c * _HN

        def tform(r, _):
            for kk in range(_CS // 16):
                v = idx3[r, pl.ds(16 * kk, 16)]
                lo = v - base
                ok = (lo >= 0) & (lo < _HN)
                idx3[r, pl.ds(16 * kk, 16)] = jnp.where(ok, lo, _TRASH)
            return _

        lax.fori_loop(0, _NCHS, tform, None)
        plsc.subcore_barrier()

        def m_start(ch, b):
            off = pl.multiple_of(eb + ch * _CS, 8)
            pltpu.make_async_copy(
                m_hbm.at[pl.ds(off, _CS)], mbufs[b], msem.at[b]).start()

        def m_wait(b):
            pltpu.make_async_copy(m_hbm.at[pl.ds(0, _CS)], mbufs[b],
                                  msem.at[b]).wait()

        for b in range(_NB - 1):
            m_start(b, b)

        def outer(i, _):
            kk = i * _NB
            for b in range(_NB):
                ch = kk + b
                b4 = (b + _NB - 1) % _NB

                @pl.when(ch + _NB - 1 < _NCHS)
                def _():
                    m_start(ch + _NB - 1, b4)

                m_wait(b)
                pltpu.sync_copy(mbufs[b], acc.at[idx3.at[ch]], add=True)
            return _

        lax.fori_loop(0, _NCHS // _NB, outer, None)
        plsc.subcore_barrier()
        pltpu.sync_copy(acc.at[pl.ds(rb, _RPT)],
                        out_hbm.at[c, pl.ds(rb, _RPT)])

    return k(m, dst3, zeros)


def kernel(x, edge_index, edge_length, forces_stack, forces_norm,
           W_e1, b_e1, W_e2, b_e2, W_e3, b_e3,
           Wf1, bf1, Ws1, bs1, Wf2, bf2, Ws2, bs2, Wf3, bf3, Ws3, bs3,
           W_d1, b_d1, W_d2, b_d2):
    xf = jnp.concatenate([x, forces_stack, forces_norm], axis=1)
    h0 = _embed(xf, W_e1, b_e1.reshape(1, -1), W_e2, b_e2.reshape(1, -1),
                W_e3, b_e3.reshape(1, -1))

    def _we(Wf, Ws):
        w = jnp.concatenate([Wf[256:], Ws[256:]], axis=1)
        return jnp.pad(w, ((0, 32 - D_EDGE), (0, 0)))
    we_all = jnp.concatenate([_we(Wf1, Ws1), _we(Wf2, Ws2), _we(Wf3, Ws3)],
                             axis=1)
    be_all = jnp.concatenate([bf1, bs1, bf2, bs2, bf3, bs3]).reshape(1, 768)
    epre = _epre(edge_length.reshape(E, 1), we_all, be_all)

    src = edge_index[0]
    dst = edge_index[1]
    dst3 = dst.reshape(_NS, _NCHS, _CS)
    zeros = jnp.zeros((_HN, 128), jnp.float32)

    h = h0
    acc = None
    layer_w = [(Wf1, Ws1), (Wf2, Ws2), (Wf3, Ws3)]
    for l in range(3):
        Wf, Ws = layer_w[l]
        ws4 = (Wf[:128], Ws[:128], Wf[128:256], Ws[128:256])
        h, td, ts = _tables(h, ws4, acc=acc, leaky=True)
        gd, gs = _sc_gather(td, ts, dst, src)
        m = _elem(gd, gs, epre, l)
        out2 = _sc_scatter(m, dst3, zeros)
        acc = out2.reshape(2 * _HN, 128)[:N]

    return _head(h, acc, W_d1, b_d1.reshape(1, -1), W_d2, b_d2.reshape(1, -1))


# speedup vs baseline: 4.0194x; 1.0920x over previous
import functools

import jax
import jax.numpy as jnp
from jax import lax
from jax.experimental import pallas as pl
from jax.experimental.pallas import tpu as pltpu
from jax.experimental.pallas import tpu_sc as plsc

N = 10000
E = 320000
D_EDGE = 26
H = 128

_NC, _NS = 2, 16
_NW = _NC * _NS
_EPW = E // _NW
_CG = 40
_NCHG = _EPW // _CG
_EPT = E // _NS
_CS = 80
_NCHS = _EPT // _CS
_NB = 5
_HN = 5120
_TRASH = _HN
_ACCR = _HN + 8
_RPT = _HN // _NS
_ZR = 8


def _leaky(x):
    return jnp.where(x >= 0, x, 0.01 * x)


def _mm(x, w):
    return jax.lax.dot_general(x, w, (((1,), (0,)), ((), ())),
                               preferred_element_type=jnp.float32)


def _softplus(x):
    return jnp.maximum(x, 0.0) + jnp.log1p(jnp.exp(-jnp.abs(x)))


def _pack_bf16(f, s):
    uf = lax.bitcast_convert_type(f, jnp.int32)
    uf = uf + 0x7FFF + (lax.shift_right_logical(uf, 16) & 1)
    us = lax.bitcast_convert_type(s, jnp.int32)
    us = us + 0x7FFF + (lax.shift_right_logical(us, 16) & 1)
    return (us & jnp.int32(-65536)) | lax.shift_right_logical(uf, 16)


def _unpack_f(w):
    return lax.bitcast_convert_type(lax.shift_left(w, 16), jnp.float32)


def _unpack_s(w):
    return lax.bitcast_convert_type(w & jnp.int32(-65536), jnp.float32)


def _embed_body(xf, w1, b1, w2, b2, w3, b3, h_out):
    h = _leaky(_mm(xf[...], w1[...]) + b1[...])
    h = _leaky(_mm(h, w2[...]) + b2[...])
    h_out[...] = _leaky(_mm(h, w3[...]) + b3[...])


def _embed(xf, w1, b1, w2, b2, w3, b3, blk=2000):
    grid = N // blk
    full = lambda a: pl.BlockSpec(a.shape, lambda i: (0, 0))
    return pl.pallas_call(
        _embed_body,
        grid=(grid,),
        in_specs=[pl.BlockSpec((blk, 128), lambda i: (i, 0)),
                  full(w1), full(b1), full(w2), full(b2), full(w3), full(b3)],
        out_specs=pl.BlockSpec((blk, 128), lambda i: (i, 0)),
        out_shape=jax.ShapeDtypeStruct((N, 128), jnp.float32),
    )(xf, w1, b1, w2, b2, w3, b3)


def _tables_body(h, wdf, wds, wsf, wss, h_out, td_out, ts_out):
    hv = h[...]
    h_out[...] = hv
    td_out[...] = _pack_bf16(_mm(hv, wdf[...]), _mm(hv, wds[...]))
    ts_out[...] = _pack_bf16(_mm(hv, wsf[...]), _mm(hv, wss[...]))


def _update_tables_body(h, a, wdf, wds, wsf, wss, h_out, td_out, ts_out,
                        *, leaky):
    hv = h[...] + a[...]
    if leaky:
        hv = _leaky(hv)
    h_out[...] = hv
    td_out[...] = _pack_bf16(_mm(hv, wdf[...]), _mm(hv, wds[...]))
    ts_out[...] = _pack_bf16(_mm(hv, wsf[...]), _mm(hv, wss[...]))


def _tables(h, ws4, acc=None, leaky=True, blk=2000):
    grid = N // blk
    full = lambda a: pl.BlockSpec(a.shape, lambda i: (0, 0))
    row = pl.BlockSpec((blk, 128), lambda i: (i, 0))
    out_shape = (jax.ShapeDtypeStruct((N, 128), jnp.float32),
                 jax.ShapeDtypeStruct((N, 128), jnp.int32),
                 jax.ShapeDtypeStruct((N, 128), jnp.int32))
    out_specs = (row, row, row)
    wspecs = [full(w) for w in ws4]
    if acc is None:
        return pl.pallas_call(
            _tables_body, grid=(grid,),
            in_specs=[row] + wspecs,
            out_specs=out_specs, out_shape=out_shape,
        )(h, *ws4)
    return pl.pallas_call(
        functools.partial(_update_tables_body, leaky=leaky), grid=(grid,),
        in_specs=[row, row] + wspecs,
        out_specs=out_specs, out_shape=out_shape,
    )(h, acc, *ws4)


def _elem_body(el, gd, gs, we, be, m_out):
    j = jax.lax.broadcasted_iota(jnp.int32, (1, 32), 1).astype(jnp.float32) * 0.2
    ea = jnp.exp(-((el[...] - j) ** 2) * 25.0)
    ep = _mm(ea, we[...]) + be[...]
    wd = gd[...]
    ws = gs[...]
    f = _unpack_f(wd) + _unpack_f(ws) + ep[:, :H]
    s = _unpack_s(wd) + _unpack_s(ws) + ep[:, H:]
    m_out[...] = jax.nn.sigmoid(f) * _softplus(s)


def _elem(el, gd, gs, we, be, blk=8000):
    grid = E // blk
    row = pl.BlockSpec((blk, 128), lambda i: (i, 0))
    full = lambda a: pl.BlockSpec(a.shape, lambda i: (0, 0))
    return pl.pallas_call(
        _elem_body,
        grid=(grid,),
        in_specs=[pl.BlockSpec((blk, 1), lambda i: (i, 0)), row, row,
                  full(we), full(be)],
        out_specs=row,
        out_shape=jax.ShapeDtypeStruct((E, H), jnp.float32),
    )(el, gd, gs, we, be)


def _head_body(h, a, wd1, bd1, wd2, bd2, out):
    hv = h[...] + a[...]
    t = jnp.maximum(_mm(hv, wd1[...]) + bd1[...], 0.0)
    out[...] = _mm(t, wd2[...]) + bd2[...]


def _head(h, acc, wd1, bd1, wd2, bd2, blk=2000):
    grid = N // blk
    full = lambda a: pl.BlockSpec(a.shape, lambda i: (0, 0))
    row = pl.BlockSpec((blk, 128), lambda i: (i, 0))
    return pl.pallas_call(
        _head_body,
        grid=(grid,),
        in_specs=[row, row, full(wd1), full(bd1), full(wd2), full(bd2)],
        out_specs=pl.BlockSpec((blk, 3), lambda i: (i, 0)),
        out_shape=jax.ShapeDtypeStruct((N, 3), jnp.float32),
    )(h, acc, wd1, bd1, wd2, bd2)


def _sc_gather(td, ts, dst, src):
    mesh = plsc.VectorSubcoreMesh(core_axis_name="c", subcore_axis_name="s")
    scratch = [
        pltpu.VMEM((_EPW,), jnp.int32),
        pltpu.VMEM((_EPW,), jnp.int32),
    ]
    scratch += [pltpu.VMEM((_CG, 128), jnp.int32) for _ in range(2 * _NB)]
    scratch += [pltpu.SemaphoreType.DMA((_NB,)) for _ in range(4)]

    @functools.partial(
        pl.kernel,
        out_type=(jax.ShapeDtypeStruct((E, 128), jnp.int32),
                  jax.ShapeDtypeStruct((E, 128), jnp.int32)),
        mesh=mesh,
        scratch_types=scratch,
    )
    def k(td_hbm, ts_hbm, dst_hbm, src_hbm, gd_hbm, gs_hbm,
          idx_d, idx_s, *bufsem):
        bufs_d = bufsem[0:_NB]
        bufs_s = bufsem[_NB:2 * _NB]
        gsem_d, gsem_s, wsem_d, wsem_s = bufsem[2 * _NB:2 * _NB + 4]
        c = lax.axis_index("c")
        s = lax.axis_index("s")
        wid = c * _NS + s
        eb = pl.multiple_of(wid * _EPW, _EPW)
        pltpu.sync_copy(dst_hbm.at[pl.ds(eb, _EPW)], idx_d)
        pltpu.sync_copy(src_hbm.at[pl.ds(eb, _EPW)], idx_s)

        def g_start(ch, b):
            off = pl.multiple_of(ch * _CG, 8)
            pltpu.make_async_copy(
                td_hbm.at[idx_d.at[pl.ds(off, _CG)]], bufs_d[b],
                gsem_d.at[b]).start()
            pltpu.make_async_copy(
                ts_hbm.at[idx_s.at[pl.ds(off, _CG)]], bufs_s[b],
                gsem_s.at[b]).start()

        def g_wait(b):
            pltpu.make_async_copy(td_hbm.at[pl.ds(0, _CG)], bufs_d[b],
                                  gsem_d.at[b]).wait()
            pltpu.make_async_copy(ts_hbm.at[pl.ds(0, _CG)], bufs_s[b],
                                  gsem_s.at[b]).wait()

        def w_start(ch, b):
            off = pl.multiple_of(eb + ch * _CG, 8)
            pltpu.make_async_copy(
                bufs_d[b], gd_hbm.at[pl.ds(off, _CG)], wsem_d.at[b]).start()
            pltpu.make_async_copy(
                bufs_s[b], gs_hbm.at[pl.ds(off, _CG)], wsem_s.at[b]).start()

        def w_wait(b):
            pltpu.make_async_copy(bufs_d[b], gd_hbm.at[pl.ds(0, _CG)],
                                  wsem_d.at[b]).wait()
            pltpu.make_async_copy(bufs_s[b], gs_hbm.at[pl.ds(0, _CG)],
                                  wsem_s.at[b]).wait()

        g_start(0, 0)
        g_start(1, 1)

        def outer(i, _):
            kk = i * _NB
            for b in range(_NB):
                ch = kk + b
                b2 = (b + 2) % _NB

                @pl.when(ch + 2 < _NCHG)
                def _():
                    @pl.when(ch >= 3)
                    def _():
                        w_wait(b2)
                    g_start(ch + 2, b2)

                g_wait(b)
                w_start(ch, b)
            return _

        lax.fori_loop(0, _NCHG // _NB, outer, None)
        for b in ((_NCHG - 3) % _NB, (_NCHG - 2) % _NB, (_NCHG - 1) % _NB):
            w_wait(b)

    return k(td, ts, dst, src)


def _sc_scatter(m, dst):
    mesh = plsc.VectorSubcoreMesh(core_axis_name="c", subcore_axis_name="s")
    scratch = [pltpu.VMEM((_ZR, 128), jnp.float32)]
    scratch += [pltpu.VMEM((_CS, 128), jnp.float32) for _ in range(_NB)]
    scratch += [pltpu.VMEM((_CS,), jnp.int32) for _ in range(_NB)]
    scratch += [pltpu.SemaphoreType.DMA((_NB,)),
                pltpu.VMEM_SHARED((_ACCR, 128), jnp.float32)]

    @functools.partial(
        pl.kernel,
        out_type=jax.ShapeDtypeStruct((2, _HN, 128), jnp.float32),
        mesh=mesh,
        scratch_types=scratch,
    )
    def k(m_hbm, dst_hbm, out_hbm, zbuf, *rest):
        mbufs = rest[0:_NB]
        ibufs = rest[_NB:2 * _NB]
        msem = rest[2 * _NB]
        acc = rest[2 * _NB + 1]
        c = lax.axis_index("c")
        s = lax.axis_index("s")
        eb = pl.multiple_of(s * _EPT, 8)
        rb = pl.multiple_of(s * _RPT, 8)

        def zrow(r, _):
            for kk in range(8):
                zbuf[r, pl.ds(16 * kk, 16)] = jnp.zeros((16,), jnp.float32)
            return _

        lax.fori_loop(0, _ZR, zrow, None)
        for t in range(_RPT // _ZR):
            pltpu.sync_copy(zbuf, acc.at[pl.ds(rb + t * _ZR, _ZR)])
        plsc.subcore_barrier()

        base = c * _HN

        def m_start(ch, b):
            off = pl.multiple_of(eb + ch * _CS, 8)
            pltpu.make_async_copy(
                m_hbm.at[pl.ds(off, _CS)], mbufs[b], msem.at[b]).start()
            pltpu.make_async_copy(
                dst_hbm.at[pl.ds(off, _CS)], ibufs[b], msem.at[b]).start()

        def m_wait(b):
            pltpu.make_async_copy(m_hbm.at[pl.ds(0, _CS)], mbufs[b],
                                  msem.at[b]).wait()
            pltpu.make_async_copy(dst_hbm.at[pl.ds(0, _CS)], ibufs[b],
                                  msem.at[b]).wait()

        for b in range(_NB - 1):
            m_start(b, b)

        def outer(i, _):
            kk = i * _NB
            for b in range(_NB):
                ch = kk + b
                b4 = (b + _NB - 1) % _NB

                @pl.when(ch + _NB - 1 < _NCHS)
                def _():
                    m_start(ch + _NB - 1, b4)

                m_wait(b)
                for kk2 in range(_CS // 16):
                    v = ibufs[b][pl.ds(16 * kk2, 16)]
                    lo = v - base
                    ok = (lo >= 0) & (lo < _HN)
                    ibufs[b][pl.ds(16 * kk2, 16)] = jnp.where(ok, lo, _TRASH)
                pltpu.sync_copy(mbufs[b], acc.at[ibufs[b]], add=True)
            return _

        lax.fori_loop(0, _NCHS // _NB, outer, None)
        plsc.subcore_barrier()
        pltpu.sync_copy(acc.at[pl.ds(rb, _RPT)],
                        out_hbm.at[c, pl.ds(rb, _RPT)])

    return k(m, dst)


def kernel(x, edge_index, edge_length, forces_stack, forces_norm,
           W_e1, b_e1, W_e2, b_e2, W_e3, b_e3,
           Wf1, bf1, Ws1, bs1, Wf2, bf2, Ws2, bs2, Wf3, bf3, Ws3, bs3,
           W_d1, b_d1, W_d2, b_d2):
    xf = jnp.concatenate([x, forces_stack, forces_norm], axis=1)
    h0 = _embed(xf, W_e1, b_e1.reshape(1, -1), W_e2, b_e2.reshape(1, -1),
                W_e3, b_e3.reshape(1, -1))

    def _we(Wf, Ws):
        w = jnp.concatenate([Wf[256:], Ws[256:]], axis=1)
        return jnp.pad(w, ((0, 32 - D_EDGE), (0, 0)))
    el2 = edge_length.reshape(E, 1)

    src = edge_index[0]
    dst = edge_index[1]

    h = h0
    acc = None
    layer_w = [(Wf1, Ws1, bf1, bs1), (Wf2, Ws2, bf2, bs2),
               (Wf3, Ws3, bf3, bs3)]
    for l in range(3):
        Wf, Ws, bfl, bsl = layer_w[l]
        ws4 = (Wf[:128], Ws[:128], Wf[128:256], Ws[128:256])
        h, td, ts = _tables(h, ws4, acc=acc, leaky=True)
        gd, gs = _sc_gather(td, ts, dst, src)
        m = _elem(el2, gd, gs, _we(Wf, Ws),
                  jnp.concatenate([bfl, bsl]).reshape(1, 256))
        acc = _sc_scatter(m, dst).reshape(2 * _HN, 128)[:N]

    return _head(h, acc, W_d1, b_d1.reshape(1, -1), W_d2, b_d2.reshape(1, -1))


# speedup vs baseline: 4.2262x; 1.0515x over previous
import functools

import jax
import jax.numpy as jnp
from jax import lax
from jax.experimental import pallas as pl
from jax.experimental.pallas import tpu as pltpu
from jax.experimental.pallas import tpu_sc as plsc

N = 10000
E = 320000
D_EDGE = 26
H = 128

_NC, _NS = 2, 16
_NW = _NC * _NS
_EH = E // 2
_EPW = _EH // _NW
_CG = 40
_NCHG = _EPW // _CG
_EPT = _EH // _NS
_CS = 80
_NCHS = _EPT // _CS
_NB = 5
_HN = 5120
_TRASH = _HN
_ACCR = _HN + 8
_RPT = _HN // _NS
_ZR = 8


def _leaky(x):
    return jnp.where(x >= 0, x, 0.01 * x)


def _mm(x, w):
    return jax.lax.dot_general(x, w, (((1,), (0,)), ((), ())),
                               preferred_element_type=jnp.float32)


def _softplus(x):
    return jnp.maximum(x, 0.0) + jnp.log1p(jnp.exp(-jnp.abs(x)))


def _pack_bf16(f, s):
    uf = lax.bitcast_convert_type(f, jnp.int32)
    uf = uf + 0x7FFF + (lax.shift_right_logical(uf, 16) & 1)
    us = lax.bitcast_convert_type(s, jnp.int32)
    us = us + 0x7FFF + (lax.shift_right_logical(us, 16) & 1)
    return (us & jnp.int32(-65536)) | lax.shift_right_logical(uf, 16)


def _unpack_f(w):
    return lax.bitcast_convert_type(lax.shift_left(w, 16), jnp.float32)


def _unpack_s(w):
    return lax.bitcast_convert_type(w & jnp.int32(-65536), jnp.float32)


def _embed_body(xf, w1, b1, w2, b2, w3, b3, h_out):
    h = _leaky(_mm(xf[...], w1[...]) + b1[...])
    h = _leaky(_mm(h, w2[...]) + b2[...])
    h_out[...] = _leaky(_mm(h, w3[...]) + b3[...])


def _embed(xf, w1, b1, w2, b2, w3, b3, blk=2000):
    grid = N // blk
    full = lambda a: pl.BlockSpec(a.shape, lambda i: (0, 0))
    return pl.pallas_call(
        _embed_body,
        grid=(grid,),
        in_specs=[pl.BlockSpec((blk, 128), lambda i: (i, 0)),
                  full(w1), full(b1), full(w2), full(b2), full(w3), full(b3)],
        out_specs=pl.BlockSpec((blk, 128), lambda i: (i, 0)),
        out_shape=jax.ShapeDtypeStruct((N, 128), jnp.float32),
    )(xf, w1, b1, w2, b2, w3, b3)


def _tables_body(h, wdf, wds, wsf, wss, h_out, td_out, ts_out):
    hv = h[...]
    h_out[...] = hv
    td_out[...] = _pack_bf16(_mm(hv, wdf[...]), _mm(hv, wds[...]))
    ts_out[...] = _pack_bf16(_mm(hv, wsf[...]), _mm(hv, wss[...]))


def _update_tables_body(h, a0, a1, wdf, wds, wsf, wss, h_out, td_out, ts_out,
                        *, leaky):
    hv = h[...] + a0[...] + a1[...]
    if leaky:
        hv = _leaky(hv)
    h_out[...] = hv
    td_out[...] = _pack_bf16(_mm(hv, wdf[...]), _mm(hv, wds[...]))
    ts_out[...] = _pack_bf16(_mm(hv, wsf[...]), _mm(hv, wss[...]))


def _tables(h, ws4, acc=None, leaky=True, blk=2000):
    grid = N // blk
    full = lambda a: pl.BlockSpec(a.shape, lambda i: (0, 0))
    row = pl.BlockSpec((blk, 128), lambda i: (i, 0))
    out_shape = (jax.ShapeDtypeStruct((N, 128), jnp.float32),
                 jax.ShapeDtypeStruct((N, 128), jnp.int32),
                 jax.ShapeDtypeStruct((N, 128), jnp.int32))
    out_specs = (row, row, row)
    wspecs = [full(w) for w in ws4]
    if acc is None:
        return pl.pallas_call(
            _tables_body, grid=(grid,),
            in_specs=[row] + wspecs,
            out_specs=out_specs, out_shape=out_shape,
        )(h, *ws4)
    return pl.pallas_call(
        functools.partial(_update_tables_body, leaky=leaky), grid=(grid,),
        in_specs=[row, row, row] + wspecs,
        out_specs=out_specs, out_shape=out_shape,
    )(h, acc[0], acc[1], *ws4)


def _elem_body(el, gd, gs, we, be, m_out):
    j = jax.lax.broadcasted_iota(jnp.int32, (1, 32), 1).astype(jnp.float32) * 0.2
    ea = jnp.exp(-((el[...] - j) ** 2) * 25.0)
    ep = _mm(ea, we[...]) + be[...]
    wd = gd[...]
    ws = gs[...]
    f = _unpack_f(wd) + _unpack_f(ws) + ep[:, :H]
    s = _unpack_s(wd) + _unpack_s(ws) + ep[:, H:]
    m_out[...] = jax.nn.sigmoid(f) * _softplus(s)


def _elem(el, gd, gs, we, be, blk=8000):
    ne = gd.shape[0]
    grid = ne // blk
    row = pl.BlockSpec((blk, 128), lambda i: (i, 0))
    full = lambda a: pl.BlockSpec(a.shape, lambda i: (0, 0))
    return pl.pallas_call(
        _elem_body,
        grid=(grid,),
        in_specs=[pl.BlockSpec((blk, 1), lambda i: (i, 0)), row, row,
                  full(we), full(be)],
        out_specs=row,
        out_shape=jax.ShapeDtypeStruct((ne, H), jnp.float32),
    )(el, gd, gs, we, be)


def _head_body(h, a0, a1, wd1, bd1, wd2, bd2, out):
    hv = h[...] + a0[...] + a1[...]
    t = jnp.maximum(_mm(hv, wd1[...]) + bd1[...], 0.0)
    out[...] = _mm(t, wd2[...]) + bd2[...]


def _head(h, acc, wd1, bd1, wd2, bd2, blk=2000):
    grid = N // blk
    full = lambda a: pl.BlockSpec(a.shape, lambda i: (0, 0))
    row = pl.BlockSpec((blk, 128), lambda i: (i, 0))
    return pl.pallas_call(
        _head_body,
        grid=(grid,),
        in_specs=[row, row, row, full(wd1), full(bd1), full(wd2), full(bd2)],
        out_specs=pl.BlockSpec((blk, 3), lambda i: (i, 0)),
        out_shape=jax.ShapeDtypeStruct((N, 3), jnp.float32),
    )(h, acc[0], acc[1], wd1, bd1, wd2, bd2)


def _sc_gather(td, ts, dst, src):
    mesh = plsc.VectorSubcoreMesh(core_axis_name="c", subcore_axis_name="s")
    scratch = [
        pltpu.VMEM((_EPW,), jnp.int32),
        pltpu.VMEM((_EPW,), jnp.int32),
    ]
    scratch += [pltpu.VMEM((_CG, 128), jnp.int32) for _ in range(2 * _NB)]
    scratch += [pltpu.SemaphoreType.DMA((_NB,)) for _ in range(4)]

    @functools.partial(
        pl.kernel,
        out_type=(jax.ShapeDtypeStruct((_EH, 128), jnp.int32),
                  jax.ShapeDtypeStruct((_EH, 128), jnp.int32)),
        mesh=mesh,
        scratch_types=scratch,
    )
    def k(td_hbm, ts_hbm, dst_hbm, src_hbm, gd_hbm, gs_hbm,
          idx_d, idx_s, *bufsem):
        bufs_d = bufsem[0:_NB]
        bufs_s = bufsem[_NB:2 * _NB]
        gsem_d, gsem_s, wsem_d, wsem_s = bufsem[2 * _NB:2 * _NB + 4]
        c = lax.axis_index("c")
        s = lax.axis_index("s")
        wid = c * _NS + s
        eb = pl.multiple_of(wid * _EPW, _EPW)
        pltpu.sync_copy(dst_hbm.at[pl.ds(eb, _EPW)], idx_d)
        pltpu.sync_copy(src_hbm.at[pl.ds(eb, _EPW)], idx_s)

        def g_start(ch, b):
            off = pl.multiple_of(ch * _CG, 8)
            pltpu.make_async_copy(
                td_hbm.at[idx_d.at[pl.ds(off, _CG)]], bufs_d[b],
                gsem_d.at[b]).start()
            pltpu.make_async_copy(
                ts_hbm.at[idx_s.at[pl.ds(off, _CG)]], bufs_s[b],
                gsem_s.at[b]).start()

        def g_wait(b):
            pltpu.make_async_copy(td_hbm.at[pl.ds(0, _CG)], bufs_d[b],
                                  gsem_d.at[b]).wait()
            pltpu.make_async_copy(ts_hbm.at[pl.ds(0, _CG)], bufs_s[b],
                                  gsem_s.at[b]).wait()

        def w_start(ch, b):
            off = pl.multiple_of(eb + ch * _CG, 8)
            pltpu.make_async_copy(
                bufs_d[b], gd_hbm.at[pl.ds(off, _CG)], wsem_d.at[b]).start()
            pltpu.make_async_copy(
                bufs_s[b], gs_hbm.at[pl.ds(off, _CG)], wsem_s.at[b]).start()

        def w_wait(b):
            pltpu.make_async_copy(bufs_d[b], gd_hbm.at[pl.ds(0, _CG)],
                                  wsem_d.at[b]).wait()
            pltpu.make_async_copy(bufs_s[b], gs_hbm.at[pl.ds(0, _CG)],
                                  wsem_s.at[b]).wait()

        g_start(0, 0)
        g_start(1, 1)

        def outer(i, _):
            kk = i * _NB
            for b in range(_NB):
                ch = kk + b
                b2 = (b + 2) % _NB

                @pl.when(ch + 2 < _NCHG)
                def _():
                    @pl.when(ch >= 3)
                    def _():
                        w_wait(b2)
                    g_start(ch + 2, b2)

                g_wait(b)
                w_start(ch, b)
            return _

        lax.fori_loop(0, _NCHG // _NB, outer, None)
        for b in ((_NCHG - 3) % _NB, (_NCHG - 2) % _NB, (_NCHG - 1) % _NB):
            w_wait(b)

    return k(td, ts, dst, src)


def _sc_scatter(m, dst):
    mesh = plsc.VectorSubcoreMesh(core_axis_name="c", subcore_axis_name="s")
    scratch = [pltpu.VMEM((_ZR, 128), jnp.float32)]
    scratch += [pltpu.VMEM((_CS, 128), jnp.float32) for _ in range(_NB)]
    scratch += [pltpu.VMEM((_CS,), jnp.int32) for _ in range(_NB)]
    scratch += [pltpu.SemaphoreType.DMA((_NB,)),
                pltpu.VMEM_SHARED((_ACCR, 128), jnp.float32)]

    @functools.partial(
        pl.kernel,
        out_type=jax.ShapeDtypeStruct((2, _HN, 128), jnp.float32),
        mesh=mesh,
        scratch_types=scratch,
    )
    def k(m_hbm, dst_hbm, out_hbm, zbuf, *rest):
        mbufs = rest[0:_NB]
        ibufs = rest[_NB:2 * _NB]
        msem = rest[2 * _NB]
        acc = rest[2 * _NB + 1]
        c = lax.axis_index("c")
        s = lax.axis_index("s")
        eb = pl.multiple_of(s * _EPT, 8)
        rb = pl.multiple_of(s * _RPT, 8)

        def zrow(r, _):
            for kk in range(8):
                zbuf[r, pl.ds(16 * kk, 16)] = jnp.zeros((16,), jnp.float32)
            return _

        lax.fori_loop(0, _ZR, zrow, None)
        for t in range(_RPT // _ZR):
            pltpu.sync_copy(zbuf, acc.at[pl.ds(rb + t * _ZR, _ZR)])
        plsc.subcore_barrier()

        base = c * _HN

        def m_start(ch, b):
            off = pl.multiple_of(eb + ch * _CS, 8)
            pltpu.make_async_copy(
                m_hbm.at[pl.ds(off, _CS)], mbufs[b], msem.at[b]).start()
            pltpu.make_async_copy(
                dst_hbm.at[pl.ds(off, _CS)], ibufs[b], msem.at[b]).start()

        def m_wait(b):
            pltpu.make_async_copy(m_hbm.at[pl.ds(0, _CS)], mbufs[b],
                                  msem.at[b]).wait()
            pltpu.make_async_copy(dst_hbm.at[pl.ds(0, _CS)], ibufs[b],
                                  msem.at[b]).wait()

        for b in range(_NB - 1):
            m_start(b, b)

        def outer(i, _):
            kk = i * _NB
            for b in range(_NB):
                ch = kk + b
                b4 = (b + _NB - 1) % _NB

                @pl.when(ch + _NB - 1 < _NCHS)
                def _():
                    m_start(ch + _NB - 1, b4)

                m_wait(b)
                for kk2 in range(_CS // 16):
                    v = ibufs[b][pl.ds(16 * kk2, 16)]
                    lo = v - base
                    ok = (lo >= 0) & (lo < _HN)
                    ibufs[b][pl.ds(16 * kk2, 16)] = jnp.where(ok, lo, _TRASH)
                pltpu.sync_copy(mbufs[b], acc.at[ibufs[b]], add=True)
            return _

        lax.fori_loop(0, _NCHS // _NB, outer, None)
        plsc.subcore_barrier()
        pltpu.sync_copy(acc.at[pl.ds(rb, _RPT)],
                        out_hbm.at[c, pl.ds(rb, _RPT)])

    return k(m, dst)


def kernel(x, edge_index, edge_length, forces_stack, forces_norm,
           W_e1, b_e1, W_e2, b_e2, W_e3, b_e3,
           Wf1, bf1, Ws1, bs1, Wf2, bf2, Ws2, bs2, Wf3, bf3, Ws3, bs3,
           W_d1, b_d1, W_d2, b_d2):
    xf = jnp.concatenate([x, forces_stack, forces_norm], axis=1)
    h0 = _embed(xf, W_e1, b_e1.reshape(1, -1), W_e2, b_e2.reshape(1, -1),
                W_e3, b_e3.reshape(1, -1))

    def _we(Wf, Ws):
        w = jnp.concatenate([Wf[256:], Ws[256:]], axis=1)
        return jnp.pad(w, ((0, 32 - D_EDGE), (0, 0)))
    el2 = edge_length.reshape(E, 1)

    src = edge_index[0]
    dst = edge_index[1]
    halves = [(dst[:_EH], src[:_EH], el2[:_EH]),
              (dst[_EH:], src[_EH:], el2[_EH:])]

    h = h0
    acc = None
    layer_w = [(Wf1, Ws1, bf1, bs1), (Wf2, Ws2, bf2, bs2),
               (Wf3, Ws3, bf3, bs3)]
    for l in range(3):
        Wf, Ws, bfl, bsl = layer_w[l]
        ws4 = (Wf[:128], Ws[:128], Wf[128:256], Ws[128:256])
        h, td, ts = _tables(h, ws4, acc=acc, leaky=True)
        we = _we(Wf, Ws)
        be = jnp.concatenate([bfl, bsl]).reshape(1, 256)
        acc = []
        for dh, sh, eh in halves:
            gd, gs = _sc_gather(td, ts, dh, sh)
            m = _elem(eh, gd, gs, we, be)
            acc.append(_sc_scatter(m, dh).reshape(2 * _HN, 128)[:N])

    return _head(h, acc, W_d1, b_d1.reshape(1, -1), W_d2, b_d2.reshape(1, -1))
